# Initial kernel scaffold; baseline (speedup 1.0000x reference)
import jax
import jax.numpy as jnp
from jax.experimental import pallas as pl


def kernel(nullary_functions, binary_weight, symmetric_weight, binary_edges, symmetric_edges):
    raise NotImplementedError("write your pallas kernel here")


# speedup vs baseline: 181.6472x; 181.6472x over previous
import functools

import jax
import jax.numpy as jnp
from jax import lax
from jax.experimental import pallas as pl
from jax.experimental.pallas import tpu as pltpu
from jax.experimental.pallas import tpu_sc as plsc

_N = 100000
_NPAD = 102400
_EBIN = 6400000
_ESYM = 1600000
_STEPS = 5
_NC = 2
_NS = 16
_CHUNK = 2000
_ZB = 800


def _make_step(n_pad, e_bin, e_sym, nc, ns, chunk, interpret=False):
  nw = nc * ns
  bin_pw = e_bin // nw
  sym_pw = e_sym // nw
  assert bin_pw % chunk == 0 and sym_pw % chunk == 0 and chunk % 16 == 0
  slice_sz = n_pad // ns
  assert slice_sz % _ZB == 0
  mesh = plsc.VectorSubcoreMesh(
      core_axis_name="c", subcore_axis_name="s",
      num_cores=nc, num_subcores=ns)

  @functools.partial(
      pl.kernel,
      out_type=jax.ShapeDtypeStruct((nc * n_pad,), jnp.float32),
      mesh=mesh,
      scratch_types=[
          pltpu.VMEM((n_pad,), jnp.float32),
          pltpu.VMEM((chunk,), jnp.int32),
          pltpu.VMEM((chunk,), jnp.int32),
          pltpu.VMEM((chunk,), jnp.int32),
          pltpu.VMEM((chunk,), jnp.float32),
          pltpu.VMEM((16,), jnp.float32),
          pltpu.VMEM_SHARED((n_pad,), jnp.float32),
      ],
      compiler_params=pltpu.CompilerParams(needs_layout_passes=False),
      interpret=interpret,
  )
  def step(probs_hbm, wb_hbm, ws_hbm, bed_hbm, sed_hbm, out_hbm,
           probs_v, dst_v, l_v, r_v, contrib_v, w_v, acc_sh):
    cid = lax.axis_index("c")
    sid = lax.axis_index("s")
    wid = sid * nc + cid

    @pl.loop(0, _ZB // 16)
    def _(i):
      contrib_v[pl.ds(i * 16, 16)] = jnp.zeros((16,), jnp.float32)
    for k in range(slice_sz // _ZB):
      pltpu.sync_copy(contrib_v.at[pl.ds(0, _ZB)],
                      acc_sh.at[pl.ds(sid * slice_sz + k * _ZB, _ZB)])

    pltpu.sync_copy(probs_hbm, probs_v)

    plsc.subcore_barrier()

    def process(ed_hbm, e_tot, per_w, w):
      base = wid * per_w

      @pl.loop(0, per_w // chunk)
      def _(j):
        off = base + j * chunk
        pltpu.sync_copy(ed_hbm.at[pl.ds(off, chunk)], dst_v)
        pltpu.sync_copy(ed_hbm.at[pl.ds(e_tot + off, chunk)], l_v)
        pltpu.sync_copy(ed_hbm.at[pl.ds(2 * e_tot + off, chunk)], r_v)

        @pl.loop(0, chunk // 16)
        def _(i):
          li = l_v[pl.ds(i * 16, 16)]
          ri = r_v[pl.ds(i * 16, 16)]
          pv = plsc.load_gather(probs_v, [li])
          qv = plsc.load_gather(probs_v, [ri])
          contrib_v[pl.ds(i * 16, 16)] = pv * qv * w

        pltpu.sync_copy(contrib_v, acc_sh.at[dst_v], add=True)

    pltpu.sync_copy(wb_hbm, w_v)
    wb = w_v[...]
    process(bed_hbm, e_bin, bin_pw, wb)
    pltpu.sync_copy(ws_hbm, w_v)
    ws = w_v[...]
    process(sed_hbm, e_sym, sym_pw, ws)

    plsc.subcore_barrier()

    pltpu.sync_copy(acc_sh.at[pl.ds(sid * slice_sz, slice_sz)],
                    out_hbm.at[pl.ds(cid * n_pad + sid * slice_sz, slice_sz)])

  return step


def _make_combine(n_pad, nc, ns, interpret=False):
  nw = nc * ns
  cs = n_pad // nw
  assert cs % 16 == 0
  mesh = plsc.VectorSubcoreMesh(
      core_axis_name="c", subcore_axis_name="s",
      num_cores=nc, num_subcores=ns)

  @functools.partial(
      pl.kernel,
      out_type=jax.ShapeDtypeStruct((n_pad,), jnp.float32),
      mesh=mesh,
      scratch_types=[
          pltpu.VMEM((cs,), jnp.float32),
          pltpu.VMEM((cs,), jnp.float32),
          pltpu.VMEM((cs,), jnp.float32),
      ],
      compiler_params=pltpu.CompilerParams(needs_layout_passes=False),
      interpret=interpret,
  )
  def combine(null_hbm, part_hbm, out_hbm, a_v, b_v, c_v):
    wid = lax.axis_index("s") * nc + lax.axis_index("c")
    base = wid * cs
    pltpu.sync_copy(null_hbm.at[pl.ds(base, cs)], a_v)
    pltpu.sync_copy(part_hbm.at[pl.ds(base, cs)], b_v)
    pltpu.sync_copy(part_hbm.at[pl.ds(n_pad + base, cs)], c_v)

    @pl.loop(0, cs // 16)
    def _(i):
      s = pl.ds(i * 16, 16)
      a_v[s] = a_v[s] + b_v[s] + c_v[s]

    pltpu.sync_copy(a_v, out_hbm.at[pl.ds(base, cs)])

  return combine


_BUILT = {}


def _get_kernels():
  if "step" not in _BUILT:
    _BUILT["step"] = _make_step(_NPAD, _EBIN, _ESYM, _NC, _NS, _CHUNK)
    _BUILT["combine"] = _make_combine(_NPAD, _NC, _NS)
  return _BUILT["step"], _BUILT["combine"]


def kernel(nullary_functions, binary_weight, symmetric_weight,
           binary_edges, symmetric_edges):
  _step, _combine = _get_kernels()
  f32 = jnp.float32
  wbv = jnp.full((16,), binary_weight, f32)
  wsv = jnp.full((16,), symmetric_weight, f32)
  null_pad = jnp.zeros((_NPAD,), f32).at[:_N].set(nullary_functions)
  bed_flat = binary_edges.reshape(-1)
  sed_flat = symmetric_edges.reshape(-1)
  probs = null_pad
  for _ in range(_STEPS):
    part = _step(probs, wbv, wsv, bed_flat, sed_flat)
    probs = _combine(null_pad, part)
  return probs[:_N]


# speedup vs baseline: 293.0716x; 1.6134x over previous
import functools

import jax
import jax.numpy as jnp
from jax import lax
from jax.experimental import pallas as pl
from jax.experimental.pallas import tpu as pltpu
from jax.experimental.pallas import tpu_sc as plsc

_N = 100000
_NPAD = 102400
_EBIN = 6400000
_ESYM = 1600000
_STEPS = 5
_NC = 2
_NS = 16
_CHUNK = 2000
_ZB = 800


def _make_step(n_pad, e_bin, e_sym, nc, ns, chunk, interpret=False):
  nw = nc * ns
  bin_pw = e_bin // nw
  sym_pw = e_sym // nw
  assert bin_pw % chunk == 0 and sym_pw % chunk == 0 and chunk % 16 == 0
  slice_sz = n_pad // ns
  assert slice_sz % _ZB == 0
  mesh = plsc.VectorSubcoreMesh(
      core_axis_name="c", subcore_axis_name="s",
      num_cores=nc, num_subcores=ns)

  @functools.partial(
      pl.kernel,
      out_type=jax.ShapeDtypeStruct((nc * n_pad,), jnp.float32),
      mesh=mesh,
      scratch_types=[
          pltpu.VMEM((_N,), jnp.float32),
          pltpu.VMEM((chunk,), jnp.int32),
          pltpu.VMEM((chunk,), jnp.int32),
          pltpu.VMEM((chunk,), jnp.int32),
          pltpu.VMEM((chunk,), jnp.int32),
          pltpu.VMEM((chunk,), jnp.int32),
          pltpu.VMEM((chunk,), jnp.int32),
          pltpu.VMEM((chunk,), jnp.float32),
          pltpu.VMEM((chunk,), jnp.float32),
          pltpu.VMEM((16,), jnp.float32),
          pltpu.VMEM((16,), jnp.float32),
          pltpu.VMEM_SHARED((n_pad,), jnp.float32),
          pltpu.SemaphoreType.DMA,
          pltpu.SemaphoreType.DMA,
      ],
      compiler_params=pltpu.CompilerParams(needs_layout_passes=False),
      interpret=interpret,
  )
  def step(probs_hbm, wb_hbm, ws_hbm, bed_hbm, sed_hbm, out_hbm,
           probs_v, d0, l0, r0, d1, l1, r1, c0, c1, wb_v, ws_v,
           acc_sh, s0, s1):
    cid = lax.axis_index("c")
    sid = lax.axis_index("s")
    wid = sid * nc + cid
    bufs = ((d0, l0, r0, c0, s0), (d1, l1, r1, c1, s1))

    @pl.loop(0, _ZB // 16)
    def _(i):
      c0[pl.ds(i * 16, 16)] = jnp.zeros((16,), jnp.float32)
    for k in range(slice_sz // _ZB):
      pltpu.sync_copy(c0.at[pl.ds(0, _ZB)],
                      acc_sh.at[pl.ds(sid * slice_sz + k * _ZB, _ZB)])

    pltpu.sync_copy(probs_hbm.at[pl.ds(0, _N)], probs_v)
    pltpu.sync_copy(wb_hbm, wb_v)
    pltpu.sync_copy(ws_hbm, ws_v)

    plsc.subcore_barrier()

    def process(ed_hbm, e_tot, per_w, w):
      base = wid * per_w
      nchunks = per_w // chunk
      npairs = nchunks // 2
      tail = nchunks - 2 * npairs

      def issue(off, k):
        d, l, r, _, s = bufs[k]
        pltpu.async_copy(ed_hbm.at[pl.ds(off, chunk)], d, s)
        pltpu.async_copy(ed_hbm.at[pl.ds(e_tot + off, chunk)], l, s)
        pltpu.async_copy(ed_hbm.at[pl.ds(2 * e_tot + off, chunk)], r, s)

      def wait_ld(k):
        d, l, r, _, s = bufs[k]
        pltpu.make_async_copy(ed_hbm.at[pl.ds(0, chunk)], d, s).wait()
        pltpu.make_async_copy(ed_hbm.at[pl.ds(0, chunk)], l, s).wait()
        pltpu.make_async_copy(ed_hbm.at[pl.ds(0, chunk)], r, s).wait()

      def do(k):
        d, l, r, c, _ = bufs[k]

        @plsc.parallel_loop(0, chunk // 16, unroll=4)
        def _(i):
          li = l[pl.ds(i * 16, 16)]
          ri = r[pl.ds(i * 16, 16)]
          pv = plsc.load_gather(probs_v, [li])
          qv = plsc.load_gather(probs_v, [ri])
          c[pl.ds(i * 16, 16)] = pv * qv * w

        pltpu.sync_copy(c, acc_sh.at[d], add=True)

      issue(base, 0)

      @pl.loop(0, npairs)
      def _(p):
        off0 = base + (2 * p) * chunk
        issue(off0 + chunk, 1)
        wait_ld(0)
        do(0)

        @pl.when(2 * p + 2 < nchunks)
        def _():
          issue(off0 + 2 * chunk, 0)

        wait_ld(1)
        do(1)

      if tail:
        wait_ld(0)
        do(0)

    wb = wb_v[...]
    process(bed_hbm, e_bin, bin_pw, wb)
    ws = ws_v[...]
    process(sed_hbm, e_sym, sym_pw, ws)

    plsc.subcore_barrier()

    pltpu.sync_copy(acc_sh.at[pl.ds(sid * slice_sz, slice_sz)],
                    out_hbm.at[pl.ds(cid * n_pad + sid * slice_sz, slice_sz)])

  return step


def _make_combine(n_pad, nc, ns, interpret=False):
  nw = nc * ns
  cs = n_pad // nw
  assert cs % 16 == 0
  mesh = plsc.VectorSubcoreMesh(
      core_axis_name="c", subcore_axis_name="s",
      num_cores=nc, num_subcores=ns)

  @functools.partial(
      pl.kernel,
      out_type=jax.ShapeDtypeStruct((n_pad,), jnp.float32),
      mesh=mesh,
      scratch_types=[
          pltpu.VMEM((cs,), jnp.float32),
          pltpu.VMEM((cs,), jnp.float32),
          pltpu.VMEM((cs,), jnp.float32),
      ],
      compiler_params=pltpu.CompilerParams(needs_layout_passes=False),
      interpret=interpret,
  )
  def combine(null_hbm, part_hbm, out_hbm, a_v, b_v, c_v):
    wid = lax.axis_index("s") * nc + lax.axis_index("c")
    base = wid * cs
    pltpu.sync_copy(null_hbm.at[pl.ds(base, cs)], a_v)
    pltpu.sync_copy(part_hbm.at[pl.ds(base, cs)], b_v)
    pltpu.sync_copy(part_hbm.at[pl.ds(n_pad + base, cs)], c_v)

    @pl.loop(0, cs // 16)
    def _(i):
      s = pl.ds(i * 16, 16)
      a_v[s] = a_v[s] + b_v[s] + c_v[s]

    pltpu.sync_copy(a_v, out_hbm.at[pl.ds(base, cs)])

  return combine


_BUILT = {}


def _get_kernels():
  if "step" not in _BUILT:
    _BUILT["step"] = _make_step(_NPAD, _EBIN, _ESYM, _NC, _NS, _CHUNK)
    _BUILT["combine"] = _make_combine(_NPAD, _NC, _NS)
  return _BUILT["step"], _BUILT["combine"]


def kernel(nullary_functions, binary_weight, symmetric_weight,
           binary_edges, symmetric_edges):
  _step, _combine = _get_kernels()
  f32 = jnp.float32
  wbv = jnp.full((16,), binary_weight, f32)
  wsv = jnp.full((16,), symmetric_weight, f32)
  null_pad = jnp.zeros((_NPAD,), f32).at[:_N].set(nullary_functions)
  bed_flat = binary_edges.reshape(-1)
  sed_flat = symmetric_edges.reshape(-1)
  probs = null_pad
  for _ in range(_STEPS):
    part = _step(probs, wbv, wsv, bed_flat, sed_flat)
    probs = _combine(null_pad, part)
  return probs[:_N]


# speedup vs baseline: 294.5290x; 1.0050x over previous
import functools

import jax
import jax.numpy as jnp
from jax import lax
from jax.experimental import pallas as pl
from jax.experimental.pallas import tpu as pltpu
from jax.experimental.pallas import tpu_sc as plsc

_N = 100000
_NPAD = 102400
_EBIN = 6400000
_ESYM = 1600000
_STEPS = 5
_NC = 2
_NS = 16
_CHUNK = 2000
_ZB = 800
_FUEL = 1 << 22


def _make_step(n_pad, e_bin, e_sym, nc, ns, chunk, interpret=False):
  nw = nc * ns
  bin_pw = e_bin // nw
  sym_pw = e_sym // nw
  assert bin_pw % chunk == 0 and sym_pw % chunk == 0 and chunk % 16 == 0
  slice_sz = n_pad // ns
  assert slice_sz % _ZB == 0
  mesh = plsc.VectorSubcoreMesh(
      core_axis_name="c", subcore_axis_name="s",
      num_cores=nc, num_subcores=ns)

  @functools.partial(
      pl.kernel,
      out_type=(
          jax.ShapeDtypeStruct((n_pad,), jnp.float32),
          jax.ShapeDtypeStruct((nc * n_pad,), jnp.float32),
          jax.ShapeDtypeStruct((nc * n_pad,), jnp.float32),
          jax.ShapeDtypeStruct((nc * 16,), jnp.int32),
      ),
      mesh=mesh,
      scratch_types=[
          pltpu.VMEM((_N,), jnp.float32),
          pltpu.VMEM((chunk,), jnp.int32),
          pltpu.VMEM((chunk,), jnp.int32),
          pltpu.VMEM((chunk,), jnp.int32),
          pltpu.VMEM((chunk,), jnp.int32),
          pltpu.VMEM((chunk,), jnp.int32),
          pltpu.VMEM((chunk,), jnp.int32),
          pltpu.VMEM((chunk,), jnp.float32),
          pltpu.VMEM((chunk,), jnp.float32),
          pltpu.VMEM((slice_sz // 2,), jnp.float32),
          pltpu.VMEM((slice_sz // 2,), jnp.float32),
          pltpu.VMEM((16,), jnp.float32),
          pltpu.VMEM((16,), jnp.float32),
          pltpu.VMEM((16,), jnp.int32),
          pltpu.VMEM((16,), jnp.int32),
          pltpu.VMEM_SHARED((n_pad,), jnp.float32),
          pltpu.SemaphoreType.DMA,
          pltpu.SemaphoreType.DMA,
      ],
      compiler_params=pltpu.CompilerParams(needs_layout_passes=False),
      interpret=interpret,
  )
  def step(null_hbm, wb_hbm, ws_hbm, bed_hbm, sed_hbm,
           out_hbm, pub_hbm, pwork_hbm, flags_hbm,
           probs_v, d0, l0, r0, d1, l1, r1, c0, c1, ta, tb,
           wb_v, ws_v, fbuf, fw, acc_sh, s0, s1):
    cid = lax.axis_index("c")
    sid = lax.axis_index("s")
    wid = sid * nc + cid
    bufs = ((d0, l0, r0, c0, s0), (d1, l1, r1, c1, s1))

    def zero_acc_slice():
      @pl.loop(0, _ZB // 16)
      def _(i):
        c0[pl.ds(i * 16, 16)] = jnp.zeros((16,), jnp.float32)
      for k in range(slice_sz // _ZB):
        pltpu.sync_copy(c0.at[pl.ds(0, _ZB)],
                        acc_sh.at[pl.ds(sid * slice_sz + k * _ZB, _ZB)])

    @pl.when(sid == 0)
    def _():
      fw[...] = jnp.zeros((16,), jnp.int32)
      pltpu.sync_copy(fw, flags_hbm.at[pl.ds(cid * 16, 16)])

    zero_acc_slice()
    pltpu.sync_copy(null_hbm.at[pl.ds(0, _N)], probs_v)
    pltpu.sync_copy(wb_hbm, wb_v)
    pltpu.sync_copy(ws_hbm, ws_v)
    wb = wb_v[...]
    ws = ws_v[...]

    plsc.subcore_barrier()

    def process(ed_hbm, e_tot, per_w, w):
      base = wid * per_w
      nchunks = per_w // chunk
      npairs = nchunks // 2
      tail = nchunks - 2 * npairs

      def issue(off, k):
        d, l, r, _, s = bufs[k]
        pltpu.async_copy(ed_hbm.at[pl.ds(off, chunk)], d, s)
        pltpu.async_copy(ed_hbm.at[pl.ds(e_tot + off, chunk)], l, s)
        pltpu.async_copy(ed_hbm.at[pl.ds(2 * e_tot + off, chunk)], r, s)

      def wait_ld(k):
        d, l, r, _, s = bufs[k]
        pltpu.make_async_copy(ed_hbm.at[pl.ds(0, chunk)], d, s).wait()
        pltpu.make_async_copy(ed_hbm.at[pl.ds(0, chunk)], l, s).wait()
        pltpu.make_async_copy(ed_hbm.at[pl.ds(0, chunk)], r, s).wait()

      def do(k):
        d, l, r, c, _ = bufs[k]

        @plsc.parallel_loop(0, chunk // 16, unroll=4)
        def _(i):
          li = l[pl.ds(i * 16, 16)]
          ri = r[pl.ds(i * 16, 16)]
          pv = plsc.load_gather(probs_v, [li])
          qv = plsc.load_gather(probs_v, [ri])
          c[pl.ds(i * 16, 16)] = pv * qv * w

        pltpu.sync_copy(c, acc_sh.at[d], add=True)

      issue(base, 0)

      @pl.loop(0, npairs)
      def _(p):
        off0 = base + (2 * p) * chunk
        issue(off0 + chunk, 1)
        wait_ld(0)
        do(0)

        @pl.when(2 * p + 2 < nchunks)
        def _():
          issue(off0 + 2 * chunk, 0)

        wait_ld(1)
        do(1)

      if tail:
        wait_ld(0)
        do(0)

    @pl.loop(0, _STEPS)
    def _(s):
      process(bed_hbm, e_bin, bin_pw, wb)
      process(sed_hbm, e_sym, sym_pw, ws)

      plsc.subcore_barrier()

      pltpu.sync_copy(
          acc_sh.at[pl.ds(sid * slice_sz, slice_sz)],
          pub_hbm.at[pl.ds(cid * n_pad + sid * slice_sz, slice_sz)])
      zero_acc_slice()

      plsc.subcore_barrier()

      @pl.when(sid == 0)
      def _():
        fw[...] = jnp.full((16,), s + 1, jnp.int32)
        pltpu.sync_copy(fw, flags_hbm.at[pl.ds(cid * 16, 16)])

      other = (cid + 1) % nc

      def poll_cond(carry):
        v, fuel = carry
        return jnp.logical_and(v < s + 1, fuel < _FUEL)

      def poll_body(carry):
        _, fuel = carry
        pltpu.sync_copy(flags_hbm.at[pl.ds(other * 16, 16)], fbuf)
        return jnp.max(fbuf[...]), fuel + 1

      lax.while_loop(poll_cond, poll_body, (jnp.int32(-1), jnp.int32(0)))

      half = slice_sz // 2
      for h in range(2):
        base = sid * slice_sz + h * half
        pltpu.sync_copy(pub_hbm.at[pl.ds(base, half)], ta)
        pltpu.sync_copy(pub_hbm.at[pl.ds(n_pad + base, half)], tb)

        @pl.loop(0, half // 16)
        def _(i):
          sl = pl.ds(i * 16, 16)
          ta[sl] = ta[sl] + tb[sl]

        pltpu.sync_copy(null_hbm.at[pl.ds(base, half)], tb)

        @pl.loop(0, half // 16)
        def _(i):
          sl = pl.ds(i * 16, 16)
          ta[sl] = ta[sl] + tb[sl]

        pltpu.sync_copy(ta, pwork_hbm.at[pl.ds(cid * n_pad + base, half)])

        @pl.when(jnp.logical_and(s == _STEPS - 1, cid == 0))
        def _():
          pltpu.sync_copy(ta, out_hbm.at[pl.ds(base, half)])

      plsc.subcore_barrier()

      @pl.when(s < _STEPS - 1)
      def _():
        pltpu.sync_copy(pwork_hbm.at[pl.ds(cid * n_pad, _N)], probs_v)

  return step


_BUILT = {}


def _get_kernels():
  if "step" not in _BUILT:
    _BUILT["step"] = _make_step(_NPAD, _EBIN, _ESYM, _NC, _NS, _CHUNK)
  return _BUILT["step"]


def kernel(nullary_functions, binary_weight, symmetric_weight,
           binary_edges, symmetric_edges):
  _step = _get_kernels()
  f32 = jnp.float32
  wbv = jnp.full((16,), binary_weight, f32)
  wsv = jnp.full((16,), symmetric_weight, f32)
  null_pad = jnp.zeros((_NPAD,), f32).at[:_N].set(nullary_functions)
  bed_flat = binary_edges.reshape(-1)
  sed_flat = symmetric_edges.reshape(-1)
  probs, _, _, _ = _step(null_pad, wbv, wsv, bed_flat, sed_flat)
  return probs[:_N]
